# baseline (speedup 1.0000x reference)
import dataclasses
import functools

import jax
import jax.numpy as jnp
from jax import lax
from jax.experimental import pallas as pl
from jax.experimental.pallas import tpu as pltpu
from jax.experimental.pallas import tpu_sc as plsc

H = 128
P = 200
NC = 2
NS = 16
NW = NC * NS
LANES = 16
W = 128


def _sc_embed(ids, pids, subword_table, positional_table, n):
    bpw = n // NW
    steps = bpw // W
    mesh = plsc.VectorSubcoreMesh(core_axis_name="c", subcore_axis_name="s")
    cp = pltpu.CompilerParams()
    if "needs_layout_passes" in pltpu.CompilerParams.__dataclass_fields__:
        cp = dataclasses.replace(cp, needs_layout_passes=False)

    @functools.partial(
        pl.kernel,
        mesh=mesh,
        compiler_params=cp,
        out_type=jax.ShapeDtypeStruct((n, H), jnp.float32),
        scratch_types=[
            pltpu.VMEM((P, H), jnp.float32),
            pltpu.VMEM((W,), jnp.int32),
            pltpu.VMEM((W,), jnp.int32),
            pltpu.VMEM((W, H), jnp.float32),
            pltpu.SemaphoreType.DMA,
        ],
    )
    def k(sub_hbm, pos_hbm, ids_hbm, pids_hbm, out_hbm,
          pos_t, ids_v, pids_v, rows_v, gsem):
        wid = lax.axis_index("s") * NC + lax.axis_index("c")
        base = wid * bpw

        pltpu.sync_copy(pos_hbm, pos_t)

        @pl.loop(0, steps)
        def _(step):
            off = base + step * W
            pltpu.sync_copy(ids_hbm.at[pl.ds(off, W)], ids_v)
            pltpu.sync_copy(pids_hbm.at[pl.ds(off, W)], pids_v)
            pltpu.async_copy(sub_hbm.at[ids_v], rows_v, gsem).wait()

            @pl.loop(0, W)
            def _(r):
                spl = plsc.load_gather(pids_v, [jnp.zeros((LANES,), jnp.int32) + r])
                for h in range(0, H, LANES):
                    col = lax.iota(jnp.int32, LANES) + h
                    chunk = plsc.load_gather(pos_t, [spl, col])
                    plsc.addupdate(rows_v.at[r, pl.ds(h, LANES)], chunk)

            pltpu.sync_copy(rows_v, out_hbm.at[pl.ds(off, W)])

    return k(subword_table, positional_table, ids, pids)


def kernel(input_ids, position_ids, subword_table, positional_table):
    b, s = input_ids.shape
    n = b * s
    out = _sc_embed(
        input_ids.reshape(n),
        position_ids.reshape(n),
        subword_table,
        positional_table,
        n,
    )
    return out.reshape(b, s, H)


# speedup vs baseline: 1.3823x; 1.3823x over previous
import dataclasses
import functools

import jax
import jax.numpy as jnp
from jax import lax
from jax.experimental import pallas as pl
from jax.experimental.pallas import tpu as pltpu
from jax.experimental.pallas import tpu_sc as plsc

H = 128
P = 200
NC = 2
NS = 16
NW = NC * NS
LANES = 16
W = 128
NBUF = 5
LA = 2


def _sc_embed(ids, pids, subword_table, positional_table, n):
    bpw = n // NW
    steps = bpw // W
    mesh = plsc.VectorSubcoreMesh(core_axis_name="c", subcore_axis_name="s")
    cp = pltpu.CompilerParams()
    if "needs_layout_passes" in pltpu.CompilerParams.__dataclass_fields__:
        cp = dataclasses.replace(cp, needs_layout_passes=False)

    @functools.partial(
        pl.kernel,
        mesh=mesh,
        compiler_params=cp,
        out_type=jax.ShapeDtypeStruct((n, H), jnp.float32),
        scratch_types=[
            pltpu.VMEM((P * H,), jnp.float32),
            pltpu.VMEM((NBUF, W), jnp.int32),
            pltpu.VMEM((NBUF, W), jnp.int32),
            pltpu.VMEM((NBUF, W, H), jnp.float32),
            pltpu.SemaphoreType.DMA((NBUF,)),
            pltpu.SemaphoreType.DMA((NBUF,)),
        ],
    )
    def k(sub_hbm, pos_hbm, ids_hbm, pids_hbm, out_hbm,
          pos_t, ids_v, pids_v, rows_v, gsem, osem):
        wid = lax.axis_index("s") * NC + lax.axis_index("c")
        base = wid * bpw

        pltpu.sync_copy(pos_hbm, pos_t)

        def stage_and_gather(step, b):
            off = base + step * W
            pltpu.sync_copy(ids_hbm.at[pl.ds(off, W)], ids_v.at[b])
            pltpu.sync_copy(pids_hbm.at[pl.ds(off, W)], pids_v.at[b])
            pltpu.make_async_copy(
                sub_hbm.at[ids_v.at[b]], rows_v.at[b], gsem.at[b]).start()

        def wait_gather(b):
            pltpu.make_async_copy(
                sub_hbm.at[ids_v.at[b]], rows_v.at[b], gsem.at[b]).wait()

        def start_writeout(step, b):
            off = base + step * W
            pltpu.make_async_copy(
                rows_v.at[b], out_hbm.at[pl.ds(off, W)], osem.at[b]).start()

        def wait_writeout(step, b):
            off = base + step * W
            pltpu.make_async_copy(
                rows_v.at[b], out_hbm.at[pl.ds(off, W)], osem.at[b]).wait()

        def accumulate(b):
            @pl.loop(0, W, step=4)
            def _(r0):
                for j in range(4):
                    r = r0 + j
                    spl = plsc.load_gather(
                        pids_v.at[b], [jnp.zeros((LANES,), jnp.int32) + r])
                    pbase = spl * H
                    for h in range(0, H, LANES):
                        idx = pbase + (lax.iota(jnp.int32, LANES) + h)
                        chunk = plsc.load_gather(pos_t, [idx])
                        plsc.addupdate(rows_v.at[b, r, pl.ds(h, LANES)], chunk)

        for s in range(LA):
            stage_and_gather(s, s)

        @pl.loop(0, steps // NBUF)
        def _(i):
            for b in range(NBUF):
                s = i * NBUF + b
                b2 = (b + LA) % NBUF
                s2 = s + LA

                @pl.when(s2 < steps)
                def _():
                    @pl.when(s2 >= NBUF)
                    def _():
                        wait_writeout(s2 - NBUF, b2)
                    stage_and_gather(s2, b2)

                wait_gather(b)
                accumulate(b)
                start_writeout(s, b)

        for b in range(NBUF):
            wait_writeout(steps - NBUF + b, b)

    return k(subword_table, positional_table.reshape(P * H), ids, pids)


def kernel(input_ids, position_ids, subword_table, positional_table):
    b, s = input_ids.shape
    n = b * s
    out = _sc_embed(
        input_ids.reshape(n),
        position_ids.reshape(n),
        subword_table,
        positional_table,
        n,
    )
    return out.reshape(b, s, H)


# speedup vs baseline: 1.6319x; 1.1805x over previous
import dataclasses
import functools

import jax
import jax.numpy as jnp
from jax import lax
from jax.experimental import pallas as pl
from jax.experimental.pallas import tpu as pltpu
from jax.experimental.pallas import tpu_sc as plsc

H = 128
NC = 2
NS = 16
NW = NC * NS
LANES = 16
W = 80
NBUF = 4
LA = 2


def _sc_embed(ids, pids, subword_table, positional_table, n):
    bpw = n // NW
    steps = bpw // W
    mesh = plsc.VectorSubcoreMesh(core_axis_name="c", subcore_axis_name="s")
    cp = pltpu.CompilerParams()
    if "needs_layout_passes" in pltpu.CompilerParams.__dataclass_fields__:
        cp = dataclasses.replace(cp, needs_layout_passes=False)

    @functools.partial(
        pl.kernel,
        mesh=mesh,
        compiler_params=cp,
        out_type=jax.ShapeDtypeStruct((n, H), jnp.float32),
        scratch_types=[
            pltpu.VMEM((NBUF, W), jnp.int32),
            pltpu.VMEM((NBUF, W), jnp.int32),
            pltpu.VMEM((NBUF, W, H), jnp.float32),
            pltpu.VMEM((NBUF, W, H), jnp.float32),
            pltpu.SemaphoreType.DMA((NBUF,)),
            pltpu.SemaphoreType.DMA((NBUF,)),
            pltpu.SemaphoreType.DMA((NBUF,)),
        ],
    )
    def k(sub_hbm, pos_hbm, ids_hbm, pids_hbm, out_hbm,
          ids_v, pids_v, rows_v, prow_v, gsem, psem, osem):
        wid = lax.axis_index("s") * NC + lax.axis_index("c")
        base = wid * bpw

        def stage_and_gather(step, b):
            off = base + step * W
            pltpu.sync_copy(ids_hbm.at[pl.ds(off, W)], ids_v.at[b])
            pltpu.sync_copy(pids_hbm.at[pl.ds(off, W)], pids_v.at[b])
            pltpu.make_async_copy(
                sub_hbm.at[ids_v.at[b]], rows_v.at[b], gsem.at[b]).start()
            pltpu.make_async_copy(
                pos_hbm.at[pids_v.at[b]], prow_v.at[b], psem.at[b]).start()

        def wait_gather(b):
            pltpu.make_async_copy(
                sub_hbm.at[ids_v.at[b]], rows_v.at[b], gsem.at[b]).wait()
            pltpu.make_async_copy(
                pos_hbm.at[pids_v.at[b]], prow_v.at[b], psem.at[b]).wait()

        def start_writeout(step, b):
            off = base + step * W
            pltpu.make_async_copy(
                rows_v.at[b], out_hbm.at[pl.ds(off, W)], osem.at[b]).start()

        def wait_writeout(step, b):
            off = base + step * W
            pltpu.make_async_copy(
                rows_v.at[b], out_hbm.at[pl.ds(off, W)], osem.at[b]).wait()

        def accumulate(b):
            @pl.loop(0, W)
            def _(r):
                for h in range(0, H, LANES):
                    sl = pl.ds(h, LANES)
                    plsc.addupdate(rows_v.at[b, r, sl], prow_v[b, r, sl])

        for s in range(LA):
            stage_and_gather(s, s)

        @pl.loop(0, steps // NBUF)
        def _(i):
            for b in range(NBUF):
                s = i * NBUF + b
                b2 = (b + LA) % NBUF
                s2 = s + LA

                @pl.when(s2 < steps)
                def _():
                    @pl.when(s2 >= NBUF)
                    def _():
                        wait_writeout(s2 - NBUF, b2)
                    stage_and_gather(s2, b2)

                wait_gather(b)
                accumulate(b)
                start_writeout(s, b)

        for b in range(NBUF):
            wait_writeout(steps - NBUF + b, b)

    return k(subword_table, positional_table, ids, pids)


def kernel(input_ids, position_ids, subword_table, positional_table):
    b, s = input_ids.shape
    n = b * s
    out = _sc_embed(
        input_ids.reshape(n),
        position_ids.reshape(n),
        subword_table,
        positional_table,
        n,
    )
    return out.reshape(b, s, H)


# speedup vs baseline: 2.5202x; 1.5444x over previous
import dataclasses
import functools

import jax
import jax.numpy as jnp
from jax import lax
from jax.experimental import pallas as pl
from jax.experimental.pallas import tpu as pltpu
from jax.experimental.pallas import tpu_sc as plsc

H = 128
P = 200
NC = 2
NS = 16
NW = NC * NS
LANES = 16
W = 80
NBUF = 4
LA = 2


def _sc_embed(ids, pids, subword_table, positional_table, n):
    bpw = n // NW
    steps = bpw // W
    mesh = plsc.VectorSubcoreMesh(core_axis_name="c", subcore_axis_name="s")
    cp = pltpu.CompilerParams()
    if "needs_layout_passes" in pltpu.CompilerParams.__dataclass_fields__:
        cp = dataclasses.replace(cp, needs_layout_passes=False)

    @functools.partial(
        pl.kernel,
        mesh=mesh,
        compiler_params=cp,
        out_type=jax.ShapeDtypeStruct((n, H), jnp.float32),
        scratch_types=[
            pltpu.VMEM((NBUF, W), jnp.int32),
            pltpu.VMEM((NBUF, W), jnp.int32),
            pltpu.VMEM((NBUF, W, H), jnp.float32),
            pltpu.VMEM((NBUF, W, H), jnp.float32),
            pltpu.VMEM((P, H), jnp.float32),
            pltpu.VMEM_SHARED((P, H), jnp.float32),
            pltpu.SemaphoreType.DMA((NBUF,)),
            pltpu.SemaphoreType.DMA((NBUF,)),
            pltpu.SemaphoreType.DMA((NBUF,)),
        ],
    )
    def k(sub_hbm, pos_hbm, ids_hbm, pids_hbm, out_hbm,
          ids_v, pids_v, rows_v, prow_v, stage_v, pos_sh, gsem, psem, osem):
        wid = lax.axis_index("s") * NC + lax.axis_index("c")
        base = wid * bpw

        @pl.when(lax.axis_index("s") == 0)
        def _():
            pltpu.sync_copy(pos_hbm, stage_v)
            pltpu.sync_copy(stage_v, pos_sh)

        plsc.subcore_barrier()

        def stage_and_gather(step, b):
            off = base + step * W
            pltpu.sync_copy(ids_hbm.at[pl.ds(off, W)], ids_v.at[b])
            pltpu.sync_copy(pids_hbm.at[pl.ds(off, W)], pids_v.at[b])
            pltpu.make_async_copy(
                sub_hbm.at[ids_v.at[b]], rows_v.at[b], gsem.at[b]).start()
            pltpu.make_async_copy(
                pos_sh.at[pids_v.at[b]], prow_v.at[b], psem.at[b]).start()

        def wait_gather(b):
            pltpu.make_async_copy(
                sub_hbm.at[ids_v.at[b]], rows_v.at[b], gsem.at[b]).wait()
            pltpu.make_async_copy(
                pos_sh.at[pids_v.at[b]], prow_v.at[b], psem.at[b]).wait()

        def start_writeout(step, b):
            off = base + step * W
            pltpu.make_async_copy(
                rows_v.at[b], out_hbm.at[pl.ds(off, W)], osem.at[b]).start()

        def wait_writeout(step, b):
            off = base + step * W
            pltpu.make_async_copy(
                rows_v.at[b], out_hbm.at[pl.ds(off, W)], osem.at[b]).wait()

        def accumulate(b):
            @pl.loop(0, W)
            def _(r):
                for h in range(0, H, LANES):
                    sl = pl.ds(h, LANES)
                    plsc.addupdate(rows_v.at[b, r, sl], prow_v[b, r, sl])

        for s in range(LA):
            stage_and_gather(s, s)

        @pl.loop(0, steps // NBUF)
        def _(i):
            for b in range(NBUF):
                s = i * NBUF + b
                b2 = (b + LA) % NBUF
                s2 = s + LA

                @pl.when(s2 < steps)
                def _():
                    @pl.when(s2 >= NBUF)
                    def _():
                        wait_writeout(s2 - NBUF, b2)
                    stage_and_gather(s2, b2)

                wait_gather(b)
                accumulate(b)
                start_writeout(s, b)

        for b in range(NBUF):
            wait_writeout(steps - NBUF + b, b)

    return k(subword_table, positional_table, ids, pids)


def kernel(input_ids, position_ids, subword_table, positional_table):
    b, s = input_ids.shape
    n = b * s
    out = _sc_embed(
        input_ids.reshape(n),
        position_ids.reshape(n),
        subword_table,
        positional_table,
        n,
    )
    return out.reshape(b, s, H)


# speedup vs baseline: 4.0075x; 1.5901x over previous
import dataclasses
import functools

import jax
import jax.numpy as jnp
from jax import lax
from jax.experimental import pallas as pl
from jax.experimental.pallas import tpu as pltpu
from jax.experimental.pallas import tpu_sc as plsc

H = 128
P = 200
NC = 2
NS = 16
NW = NC * NS
W = 128
NBUF = 5
LA = 3


def _sc_embed(ids, pids, subword_table, positional_table, n):
    bpw = n // NW
    steps = bpw // W
    mesh = plsc.VectorSubcoreMesh(core_axis_name="c", subcore_axis_name="s")
    cp = pltpu.CompilerParams()
    if "needs_layout_passes" in pltpu.CompilerParams.__dataclass_fields__:
        cp = dataclasses.replace(cp, needs_layout_passes=False)

    @functools.partial(
        pl.kernel,
        mesh=mesh,
        compiler_params=cp,
        out_type=jax.ShapeDtypeStruct((n, H), jnp.float32),
        scratch_types=[
            pltpu.VMEM((NBUF, W), jnp.int32),
            pltpu.VMEM((NBUF, W), jnp.int32),
            pltpu.VMEM((NBUF, W, H), jnp.float32),
            pltpu.VMEM((P, H), jnp.float32),
            pltpu.VMEM_SHARED((P, H), jnp.float32),
            pltpu.SemaphoreType.DMA((NBUF,)),
            pltpu.SemaphoreType.DMA((NBUF,)),
            pltpu.SemaphoreType.DMA((NBUF,)),
        ],
    )
    def k(sub_hbm, pos_hbm, ids_hbm, pids_hbm, out_hbm,
          ids_v, pids_v, rows_v, stage_v, pos_sh, gsem, psem, osem):
        wid = lax.axis_index("s") * NC + lax.axis_index("c")
        base = wid * bpw

        @pl.when(lax.axis_index("s") == 0)
        def _():
            pltpu.sync_copy(pos_hbm, stage_v)
            pltpu.sync_copy(stage_v, pos_sh)

        plsc.subcore_barrier()

        def stage_and_gather(step, b):
            off = base + step * W
            pltpu.sync_copy(ids_hbm.at[pl.ds(off, W)], ids_v.at[b])
            pltpu.sync_copy(pids_hbm.at[pl.ds(off, W)], pids_v.at[b])
            pltpu.make_async_copy(
                sub_hbm.at[ids_v.at[b]], rows_v.at[b], gsem.at[b]).start()

        def start_posadd(b):
            pltpu.make_async_copy(
                sub_hbm.at[ids_v.at[b]], rows_v.at[b], gsem.at[b]).wait()
            pltpu.async_copy(
                pos_sh.at[pids_v.at[b]], rows_v.at[b], psem.at[b], add=True)

        def wait_posadd(b):
            pltpu.make_async_copy(
                pos_sh.at[pids_v.at[b]], rows_v.at[b], psem.at[b]).wait()

        def start_writeout(step, b):
            off = base + step * W
            pltpu.make_async_copy(
                rows_v.at[b], out_hbm.at[pl.ds(off, W)], osem.at[b]).start()

        def wait_writeout(step, b):
            off = base + step * W
            pltpu.make_async_copy(
                rows_v.at[b], out_hbm.at[pl.ds(off, W)], osem.at[b]).wait()

        for s in range(LA):
            stage_and_gather(s, s)
        start_posadd(0)

        @pl.loop(0, steps // NBUF)
        def _(i):
            for b in range(NBUF):
                s = i * NBUF + b
                b2 = (b + LA) % NBUF
                s2 = s + LA
                b1 = (b + 1) % NBUF
                s1 = s + 1

                @pl.when(s2 < steps)
                def _():
                    @pl.when(s2 >= NBUF)
                    def _():
                        wait_writeout(s2 - NBUF, b2)
                    stage_and_gather(s2, b2)

                @pl.when(s1 < steps)
                def _():
                    start_posadd(b1)

                wait_posadd(b)
                start_writeout(s, b)

        for b in range(NBUF):
            wait_writeout(steps - NBUF + b, b)

    return k(subword_table, positional_table, ids, pids)


def kernel(input_ids, position_ids, subword_table, positional_table):
    b, s = input_ids.shape
    n = b * s
    out = _sc_embed(
        input_ids.reshape(n),
        position_ids.reshape(n),
        subword_table,
        positional_table,
        n,
    )
    return out.reshape(b, s, H)


# speedup vs baseline: 4.1476x; 1.0350x over previous
import dataclasses
import functools

import jax
import jax.numpy as jnp
from jax import lax
from jax.experimental import pallas as pl
from jax.experimental.pallas import tpu as pltpu
from jax.experimental.pallas import tpu_sc as plsc

H = 128
P = 200
NC = 2
NS = 16
NW = NC * NS
W = 128
NBUF = 5
LA = 3


def _sc_embed(ids, pids, subword_table, positional_table, n):
    bpw = n // NW
    steps = bpw // W
    mesh = plsc.VectorSubcoreMesh(core_axis_name="c", subcore_axis_name="s")
    cp = pltpu.CompilerParams()
    if "needs_layout_passes" in pltpu.CompilerParams.__dataclass_fields__:
        cp = dataclasses.replace(cp, needs_layout_passes=False)

    @functools.partial(
        pl.kernel,
        mesh=mesh,
        compiler_params=cp,
        out_type=jax.ShapeDtypeStruct((n, H), jnp.float32),
        scratch_types=[
            pltpu.VMEM((bpw,), jnp.int32),
            pltpu.VMEM((bpw,), jnp.int32),
            pltpu.VMEM((NBUF, W, H), jnp.float32),
            pltpu.VMEM((P, H), jnp.float32),
            pltpu.VMEM_SHARED((P, H), jnp.float32),
            pltpu.SemaphoreType.DMA((NBUF,)),
            pltpu.SemaphoreType.DMA((NBUF,)),
            pltpu.SemaphoreType.DMA((NBUF,)),
        ],
    )
    def k(sub_hbm, pos_hbm, ids_hbm, pids_hbm, out_hbm,
          ids_v, pids_v, rows_v, stage_v, pos_sh, gsem, psem, osem):
        wid = lax.axis_index("s") * NC + lax.axis_index("c")
        base = wid * bpw

        @pl.when(lax.axis_index("s") == 0)
        def _():
            pltpu.sync_copy(pos_hbm, stage_v)
            pltpu.sync_copy(stage_v, pos_sh)

        pltpu.sync_copy(ids_hbm.at[pl.ds(base, bpw)], ids_v)
        pltpu.sync_copy(pids_hbm.at[pl.ds(base, bpw)], pids_v)
        plsc.subcore_barrier()

        def stage_and_gather(step, b):
            pltpu.make_async_copy(
                sub_hbm.at[ids_v.at[pl.ds(step * W, W)]], rows_v.at[b],
                gsem.at[b]).start()

        def start_posadd(step, b):
            pltpu.make_async_copy(
                sub_hbm.at[ids_v.at[pl.ds(step * W, W)]], rows_v.at[b],
                gsem.at[b]).wait()
            pltpu.async_copy(
                pos_sh.at[pids_v.at[pl.ds(step * W, W)]], rows_v.at[b],
                psem.at[b], add=True)

        def wait_posadd(step, b):
            pltpu.make_async_copy(
                pos_sh.at[pids_v.at[pl.ds(step * W, W)]], rows_v.at[b],
                psem.at[b]).wait()

        def start_writeout(step, b):
            off = base + step * W
            pltpu.make_async_copy(
                rows_v.at[b], out_hbm.at[pl.ds(off, W)], osem.at[b]).start()

        def wait_writeout(step, b):
            off = base + step * W
            pltpu.make_async_copy(
                rows_v.at[b], out_hbm.at[pl.ds(off, W)], osem.at[b]).wait()

        for s in range(LA):
            stage_and_gather(s, s)
        start_posadd(0, 0)

        @pl.loop(0, steps // NBUF)
        def _(i):
            for b in range(NBUF):
                s = i * NBUF + b
                b2 = (b + LA) % NBUF
                s2 = s + LA
                b1 = (b + 1) % NBUF
                s1 = s + 1

                @pl.when(s2 < steps)
                def _():
                    @pl.when(s2 >= NBUF)
                    def _():
                        wait_writeout(s2 - NBUF, b2)
                    stage_and_gather(s2, b2)

                @pl.when(s1 < steps)
                def _():
                    start_posadd(s1, b1)

                wait_posadd(s, b)
                start_writeout(s, b)

        for b in range(NBUF):
            wait_writeout(steps - NBUF + b, b)

    return k(subword_table, positional_table, ids, pids)


def kernel(input_ids, position_ids, subword_table, positional_table):
    b, s = input_ids.shape
    n = b * s
    out = _sc_embed(
        input_ids.reshape(n),
        position_ids.reshape(n),
        subword_table,
        positional_table,
        n,
    )
    return out.reshape(b, s, H)


# speedup vs baseline: 4.1756x; 1.0068x over previous
import dataclasses
import functools

import jax
import jax.numpy as jnp
from jax import lax
from jax.experimental import pallas as pl
from jax.experimental.pallas import tpu as pltpu
from jax.experimental.pallas import tpu_sc as plsc

H = 128
P = 200
NC = 2
NS = 16
NW = NC * NS
W = 128
NBUF = 5
LA = 4


def _sc_embed(ids, pids, subword_table, positional_table, n):
    bpw = n // NW
    steps = bpw // W
    mesh = plsc.VectorSubcoreMesh(core_axis_name="c", subcore_axis_name="s")
    cp = pltpu.CompilerParams()
    if "needs_layout_passes" in pltpu.CompilerParams.__dataclass_fields__:
        cp = dataclasses.replace(cp, needs_layout_passes=False)

    @functools.partial(
        pl.kernel,
        mesh=mesh,
        compiler_params=cp,
        out_type=jax.ShapeDtypeStruct((n, H), jnp.float32),
        scratch_types=[
            pltpu.VMEM((bpw,), jnp.int32),
            pltpu.VMEM((bpw,), jnp.int32),
            pltpu.VMEM((NBUF, W, H), jnp.float32),
            pltpu.VMEM((P, H), jnp.float32),
            pltpu.VMEM_SHARED((P, H), jnp.float32),
            pltpu.SemaphoreType.DMA((NBUF,)),
            pltpu.SemaphoreType.DMA((NBUF,)),
            pltpu.SemaphoreType.DMA((NBUF,)),
        ],
    )
    def k(sub_hbm, pos_hbm, ids_hbm, pids_hbm, out_hbm,
          ids_v, pids_v, rows_v, stage_v, pos_sh, gsem, psem, osem):
        wid = lax.axis_index("s") * NC + lax.axis_index("c")
        base = wid * bpw

        @pl.when(lax.axis_index("s") == 0)
        def _():
            pltpu.sync_copy(pos_hbm, stage_v)
            pltpu.sync_copy(stage_v, pos_sh)

        pltpu.sync_copy(ids_hbm.at[pl.ds(base, bpw)], ids_v)
        pltpu.sync_copy(pids_hbm.at[pl.ds(base, bpw)], pids_v)

        def stage_and_gather(step, b):
            pltpu.make_async_copy(
                sub_hbm.at[ids_v.at[pl.ds(step * W, W)]], rows_v.at[b],
                gsem.at[b]).start()

        def start_posadd(step, b):
            pltpu.make_async_copy(
                sub_hbm.at[ids_v.at[pl.ds(step * W, W)]], rows_v.at[b],
                gsem.at[b]).wait()
            pltpu.async_copy(
                pos_sh.at[pids_v.at[pl.ds(step * W, W)]], rows_v.at[b],
                psem.at[b], add=True)

        def wait_posadd(step, b):
            pltpu.make_async_copy(
                pos_sh.at[pids_v.at[pl.ds(step * W, W)]], rows_v.at[b],
                psem.at[b]).wait()

        def start_writeout(step, b):
            off = base + step * W
            pltpu.make_async_copy(
                rows_v.at[b], out_hbm.at[pl.ds(off, W)], osem.at[b]).start()

        def wait_writeout(step, b):
            off = base + step * W
            pltpu.make_async_copy(
                rows_v.at[b], out_hbm.at[pl.ds(off, W)], osem.at[b]).wait()

        for s in range(LA):
            stage_and_gather(s, s)
        plsc.subcore_barrier()
        start_posadd(0, 0)

        @pl.loop(0, steps // NBUF)
        def _(i):
            for b in range(NBUF):
                s = i * NBUF + b
                b2 = (b + LA) % NBUF
                s2 = s + LA
                b1 = (b + 1) % NBUF
                s1 = s + 1

                @pl.when(s2 < steps)
                def _():
                    @pl.when(s2 >= NBUF)
                    def _():
                        wait_writeout(s2 - NBUF, b2)
                    stage_and_gather(s2, b2)

                @pl.when(s1 < steps)
                def _():
                    start_posadd(s1, b1)

                wait_posadd(s, b)
                start_writeout(s, b)

        for b in range(NBUF):
            wait_writeout(steps - NBUF + b, b)

    return k(subword_table, positional_table, ids, pids)


def kernel(input_ids, position_ids, subword_table, positional_table):
    b, s = input_ids.shape
    n = b * s
    out = _sc_embed(
        input_ids.reshape(n),
        position_ids.reshape(n),
        subword_table,
        positional_table,
        n,
    )
    return out.reshape(b, s, H)


# speedup vs baseline: 4.2124x; 1.0088x over previous
import dataclasses
import functools

import jax
import jax.numpy as jnp
from jax import lax
from jax.experimental import pallas as pl
from jax.experimental.pallas import tpu as pltpu
from jax.experimental.pallas import tpu_sc as plsc

H = 128
P = 200
NC = 2
NS = 16
NW = NC * NS
W = 64
NBUF = 10
LA = 6


def _sc_embed(ids, pids, subword_table, positional_table, n):
    bpw = n // NW
    steps = bpw // W
    mesh = plsc.VectorSubcoreMesh(core_axis_name="c", subcore_axis_name="s")
    cp = pltpu.CompilerParams()
    if "needs_layout_passes" in pltpu.CompilerParams.__dataclass_fields__:
        cp = dataclasses.replace(cp, needs_layout_passes=False)

    @functools.partial(
        pl.kernel,
        mesh=mesh,
        compiler_params=cp,
        out_type=jax.ShapeDtypeStruct((n, H), jnp.float32),
        scratch_types=[
            pltpu.VMEM((bpw,), jnp.int32),
            pltpu.VMEM((bpw,), jnp.int32),
            pltpu.VMEM((NBUF, W, H), jnp.float32),
            pltpu.VMEM((P, H), jnp.float32),
            pltpu.VMEM_SHARED((P, H), jnp.float32),
            pltpu.SemaphoreType.DMA((NBUF,)),
            pltpu.SemaphoreType.DMA((NBUF,)),
            pltpu.SemaphoreType.DMA((NBUF,)),
        ],
    )
    def k(sub_hbm, pos_hbm, ids_hbm, pids_hbm, out_hbm,
          ids_v, pids_v, rows_v, stage_v, pos_sh, gsem, psem, osem):
        wid = lax.axis_index("s") * NC + lax.axis_index("c")
        base = wid * bpw

        @pl.when(lax.axis_index("s") == 0)
        def _():
            pltpu.sync_copy(pos_hbm, stage_v)
            pltpu.sync_copy(stage_v, pos_sh)

        pltpu.sync_copy(ids_hbm.at[pl.ds(base, bpw)], ids_v)
        pltpu.sync_copy(pids_hbm.at[pl.ds(base, bpw)], pids_v)

        def stage_and_gather(step, b):
            pltpu.make_async_copy(
                sub_hbm.at[ids_v.at[pl.ds(step * W, W)]], rows_v.at[b],
                gsem.at[b]).start()

        def start_posadd(step, b):
            pltpu.make_async_copy(
                sub_hbm.at[ids_v.at[pl.ds(step * W, W)]], rows_v.at[b],
                gsem.at[b]).wait()
            pltpu.async_copy(
                pos_sh.at[pids_v.at[pl.ds(step * W, W)]], rows_v.at[b],
                psem.at[b], add=True)

        def wait_posadd(step, b):
            pltpu.make_async_copy(
                pos_sh.at[pids_v.at[pl.ds(step * W, W)]], rows_v.at[b],
                psem.at[b]).wait()

        def start_writeout(step, b):
            off = base + step * W
            pltpu.make_async_copy(
                rows_v.at[b], out_hbm.at[pl.ds(off, W)], osem.at[b]).start()

        def wait_writeout(step, b):
            off = base + step * W
            pltpu.make_async_copy(
                rows_v.at[b], out_hbm.at[pl.ds(off, W)], osem.at[b]).wait()

        for s in range(LA):
            stage_and_gather(s, s)
        plsc.subcore_barrier()
        start_posadd(0, 0)

        @pl.loop(0, steps // NBUF)
        def _(i):
            for b in range(NBUF):
                s = i * NBUF + b
                b2 = (b + LA) % NBUF
                s2 = s + LA
                b1 = (b + 1) % NBUF
                s1 = s + 1

                @pl.when(s2 < steps)
                def _():
                    @pl.when(s2 >= NBUF)
                    def _():
                        wait_writeout(s2 - NBUF, b2)
                    stage_and_gather(s2, b2)

                @pl.when(s1 < steps)
                def _():
                    start_posadd(s1, b1)

                wait_posadd(s, b)
                start_writeout(s, b)

        for b in range(NBUF):
            wait_writeout(steps - NBUF + b, b)

    return k(subword_table, positional_table, ids, pids)


def kernel(input_ids, position_ids, subword_table, positional_table):
    b, s = input_ids.shape
    n = b * s
    out = _sc_embed(
        input_ids.reshape(n),
        position_ids.reshape(n),
        subword_table,
        positional_table,
        n,
    )
    return out.reshape(b, s, H)


# speedup vs baseline: 4.2138x; 1.0003x over previous
import dataclasses
import functools

import jax
import jax.numpy as jnp
from jax import lax
from jax.experimental import pallas as pl
from jax.experimental.pallas import tpu as pltpu
from jax.experimental.pallas import tpu_sc as plsc

H = 128
P = 200
NC = 2
NS = 16
NW = NC * NS
W = 64
NBUF = 10
LA = 8


def _sc_embed(ids, pids, subword_table, positional_table, n):
    bpw = n // NW
    steps = bpw // W
    mesh = plsc.VectorSubcoreMesh(core_axis_name="c", subcore_axis_name="s")
    cp = pltpu.CompilerParams()
    if "needs_layout_passes" in pltpu.CompilerParams.__dataclass_fields__:
        cp = dataclasses.replace(cp, needs_layout_passes=False)

    @functools.partial(
        pl.kernel,
        mesh=mesh,
        compiler_params=cp,
        out_type=jax.ShapeDtypeStruct((n, H), jnp.float32),
        scratch_types=[
            pltpu.VMEM((bpw,), jnp.int32),
            pltpu.VMEM((bpw,), jnp.int32),
            pltpu.VMEM((NBUF, W, H), jnp.float32),
            pltpu.VMEM((P, H), jnp.float32),
            pltpu.VMEM_SHARED((P, H), jnp.float32),
            pltpu.SemaphoreType.DMA((NBUF,)),
            pltpu.SemaphoreType.DMA((NBUF,)),
            pltpu.SemaphoreType.DMA((NBUF,)),
        ],
    )
    def k(sub_hbm, pos_hbm, ids_hbm, pids_hbm, out_hbm,
          ids_v, pids_v, rows_v, stage_v, pos_sh, gsem, psem, osem):
        wid = lax.axis_index("s") * NC + lax.axis_index("c")
        base = wid * bpw

        @pl.when(lax.axis_index("s") == 0)
        def _():
            pltpu.sync_copy(pos_hbm, stage_v)
            pltpu.sync_copy(stage_v, pos_sh)

        pltpu.sync_copy(ids_hbm.at[pl.ds(base, bpw)], ids_v)
        pltpu.sync_copy(pids_hbm.at[pl.ds(base, bpw)], pids_v)

        def stage_and_gather(step, b):
            pltpu.make_async_copy(
                sub_hbm.at[ids_v.at[pl.ds(step * W, W)]], rows_v.at[b],
                gsem.at[b]).start()

        def start_posadd(step, b):
            pltpu.make_async_copy(
                sub_hbm.at[ids_v.at[pl.ds(step * W, W)]], rows_v.at[b],
                gsem.at[b]).wait()
            pltpu.async_copy(
                pos_sh.at[pids_v.at[pl.ds(step * W, W)]], rows_v.at[b],
                psem.at[b], add=True)

        def wait_posadd(step, b):
            pltpu.make_async_copy(
                pos_sh.at[pids_v.at[pl.ds(step * W, W)]], rows_v.at[b],
                psem.at[b]).wait()

        def start_writeout(step, b):
            off = base + step * W
            pltpu.make_async_copy(
                rows_v.at[b], out_hbm.at[pl.ds(off, W)], osem.at[b]).start()

        def wait_writeout(step, b):
            off = base + step * W
            pltpu.make_async_copy(
                rows_v.at[b], out_hbm.at[pl.ds(off, W)], osem.at[b]).wait()

        for s in range(LA):
            stage_and_gather(s, s)
        plsc.subcore_barrier()
        start_posadd(0, 0)

        @pl.loop(0, steps // NBUF)
        def _(i):
            for b in range(NBUF):
                s = i * NBUF + b
                b2 = (b + LA) % NBUF
                s2 = s + LA
                b1 = (b + 1) % NBUF
                s1 = s + 1

                @pl.when(s2 < steps)
                def _():
                    @pl.when(s2 >= NBUF)
                    def _():
                        wait_writeout(s2 - NBUF, b2)
                    stage_and_gather(s2, b2)

                @pl.when(s1 < steps)
                def _():
                    start_posadd(s1, b1)

                wait_posadd(s, b)
                start_writeout(s, b)

        for b in range(NBUF):
            wait_writeout(steps - NBUF + b, b)

    return k(subword_table, positional_table, ids, pids)


def kernel(input_ids, position_ids, subword_table, positional_table):
    b, s = input_ids.shape
    n = b * s
    out = _sc_embed(
        input_ids.reshape(n),
        position_ids.reshape(n),
        subword_table,
        positional_table,
        n,
    )
    return out.reshape(b, s, H)
